# Initial kernel scaffold; baseline (speedup 1.0000x reference)
import jax
import jax.numpy as jnp
from jax.experimental import pallas as pl


def kernel(x, edge_index, edge_attr, batch, W_i, W_h, W_o, ffn_w1, ffn_w2):
    raise NotImplementedError("write your pallas kernel here")


# speedup vs baseline: 3.0997x; 3.0997x over previous
import jax
import jax.numpy as jnp
from jax import lax
from jax.experimental import pallas as pl
from jax.experimental.pallas import tpu as pltpu
from jax.experimental.pallas import tpu_sc as plsc

NC = 2
NS = 16
NW = NC * NS
G = 64


def _sc_mesh():
    return plsc.VectorSubcoreMesh(
        core_axis_name="c", subcore_axis_name="s", num_cores=NC, num_subcores=NS
    )


def _chunk(per_w):
    for c in range(min(200, per_w), 7, -1):
        if per_w % c == 0 and c % 8 == 0:
            return c
    raise ValueError(f"no valid chunk for {per_w}")


def _sc_gather(table, idx):
    NP, H = table.shape
    E = idx.shape[0]
    per_w = E // NW
    C = _chunk(per_w)
    stripe = NP // NS

    def body(table_hbm, idx_hbm, out_hbm, s_sh, idx_v, rows_v, sem):
        cid = lax.axis_index("c")
        sid = lax.axis_index("s")
        wid = sid * NC + cid
        pltpu.sync_copy(table_hbm.at[pl.ds(sid * stripe, stripe)],
                        s_sh.at[pl.ds(sid * stripe, stripe)])
        plsc.subcore_barrier()

        def step(i, carry):
            base = wid * per_w + i * C
            pltpu.sync_copy(idx_hbm.at[pl.ds(base, C)], idx_v)
            pltpu.async_copy(s_sh.at[idx_v], rows_v, sem).wait()
            pltpu.sync_copy(rows_v, out_hbm.at[pl.ds(base, C)])
            return carry

        lax.fori_loop(0, per_w // C, step, 0)

    return pl.kernel(
        body,
        out_type=jax.ShapeDtypeStruct((E, H), jnp.float32),
        mesh=_sc_mesh(),
        scratch_types=[
            pltpu.VMEM_SHARED((NP, H), jnp.float32),
            pltpu.VMEM((C,), jnp.int32),
            pltpu.VMEM((C, H), jnp.float32),
            pltpu.SemaphoreType.DMA,
        ],
    )(table, idx)


def _sc_scatter(vals, dst, zeros):
    E, H = vals.shape
    NP = zeros.shape[0]
    per_w = E // NW
    C = _chunk(per_w)
    stripe = NP // NS

    def body(vals_hbm, dst_hbm, zero_hbm, out_hbm, s_sh, idx_v, vals_v):
        cid = lax.axis_index("c")
        sid = lax.axis_index("s")
        wid = sid * NC + cid
        pltpu.sync_copy(zero_hbm.at[pl.ds(sid * stripe, stripe)],
                        s_sh.at[pl.ds(sid * stripe, stripe)])
        plsc.subcore_barrier()

        def sstep(i, carry):
            base = wid * per_w + i * C
            pltpu.sync_copy(dst_hbm.at[pl.ds(base, C)], idx_v)
            pltpu.sync_copy(vals_hbm.at[pl.ds(base, C)], vals_v)
            pltpu.sync_copy(vals_v, s_sh.at[idx_v], add=True)
            return carry

        lax.fori_loop(0, per_w // C, sstep, 0)
        plsc.subcore_barrier()
        pltpu.sync_copy(s_sh.at[pl.ds(sid * stripe, stripe)],
                        out_hbm.at[cid, pl.ds(sid * stripe, stripe)])

    return pl.kernel(
        body,
        out_type=jax.ShapeDtypeStruct((NC, NP, H), jnp.float32),
        mesh=_sc_mesh(),
        scratch_types=[
            pltpu.VMEM_SHARED((NP, H), jnp.float32),
            pltpu.VMEM((C,), jnp.int32),
            pltpu.VMEM((C, H), jnp.float32),
        ],
    )(vals, dst, zeros)


def _tc_node_pre(x_p, W1):
    NP, _ = x_p.shape
    H = W1.shape[1]

    def body(x_ref, w_ref, out_ref):
        out_ref[...] = jnp.dot(x_ref[...], w_ref[...],
                               preferred_element_type=jnp.float32)

    return pl.pallas_call(
        body, out_shape=jax.ShapeDtypeStruct((NP, H), jnp.float32)
    )(x_p, W1)


def _tc_merge(parts):
    _, NP, H = parts.shape

    def body(p_ref, out_ref):
        out_ref[...] = p_ref[0] + p_ref[1]

    return pl.pallas_call(
        body, out_shape=jax.ShapeDtypeStruct((NP, H), jnp.float32)
    )(parts)


def _tc_edge_init(xs, ea, W2, Wh, BE):
    E, H = xs.shape
    DE = ea.shape[1]

    def body(xs_ref, ea_ref, w2_ref, wh_ref, h0_ref, g_ref):
        h0 = jax.nn.relu(
            xs_ref[...]
            + jnp.dot(ea_ref[...], w2_ref[...], preferred_element_type=jnp.float32)
        )
        h0_ref[...] = h0
        g_ref[...] = jnp.dot(h0, wh_ref[...], preferred_element_type=jnp.float32)

    return pl.pallas_call(
        body,
        grid=(E // BE,),
        in_specs=[
            pl.BlockSpec((BE, H), lambda i: (i, 0)),
            pl.BlockSpec((BE, DE), lambda i: (i, 0)),
            pl.BlockSpec((DE, H), lambda i: (0, 0)),
            pl.BlockSpec((H, H), lambda i: (0, 0)),
        ],
        out_specs=[
            pl.BlockSpec((BE, H), lambda i: (i, 0)),
            pl.BlockSpec((BE, H), lambda i: (i, 0)),
        ],
        out_shape=[
            jax.ShapeDtypeStruct((E, H), jnp.float32),
            jax.ShapeDtypeStruct((E, H), jnp.float32),
        ],
    )(xs, ea, W2, Wh)


def _pair_swap(g):
    up = jnp.roll(g, -1, axis=0)
    down = jnp.roll(g, 1, axis=0)
    par = lax.broadcasted_iota(jnp.int32, g.shape, 0) % 2
    return jnp.where(par == 0, up, down)


def _tc_edge_step(h0, ss, g, Wh, BE, last):
    E, H = h0.shape

    def body(h0_ref, ss_ref, g_ref, wh_ref, out_ref):
        h = jax.nn.relu(h0_ref[...] + ss_ref[...] - _pair_swap(g_ref[...]))
        if not last:
            h = jnp.dot(h, wh_ref[...], preferred_element_type=jnp.float32)
        out_ref[...] = h

    return pl.pallas_call(
        body,
        grid=(E // BE,),
        in_specs=[
            pl.BlockSpec((BE, H), lambda i: (i, 0)),
            pl.BlockSpec((BE, H), lambda i: (i, 0)),
            pl.BlockSpec((BE, H), lambda i: (i, 0)),
            pl.BlockSpec((H, H), lambda i: (0, 0)),
        ],
        out_specs=pl.BlockSpec((BE, H), lambda i: (i, 0)),
        out_shape=jax.ShapeDtypeStruct((E, H), jnp.float32),
    )(h0, ss, g, Wh)


def _tc_head(x_p, a2, batch2, Wo1, Wo2, f1, f2):
    NP, _ = x_p.shape

    def body(x_ref, a_ref, b_ref, wo1_ref, wo2_ref, f1_ref, f2_ref, out_ref):
        a = a_ref[0] + a_ref[1]
        nh = jax.nn.relu(
            jnp.dot(x_ref[...], wo1_ref[...], preferred_element_type=jnp.float32)
            + jnp.dot(a, wo2_ref[...], preferred_element_type=jnp.float32)
        )
        gid = lax.broadcasted_iota(jnp.int32, (G, NP), 0)
        oh = (b_ref[...] == gid).astype(jnp.float32)
        cnt = jnp.maximum(jnp.sum(oh, axis=1), 1.0)
        gh = jnp.dot(oh, nh, preferred_element_type=jnp.float32) / cnt[:, None]
        hid = jax.nn.relu(jnp.dot(gh, f1_ref[...], preferred_element_type=jnp.float32))
        out_ref[...] = jnp.dot(hid, f2_ref[...], preferred_element_type=jnp.float32)

    return pl.pallas_call(
        body, out_shape=jax.ShapeDtypeStruct((G, 1), jnp.float32)
    )(x_p, a2, batch2, Wo1, Wo2, f1, f2)


def kernel(x, edge_index, edge_attr, batch, W_i, W_h, W_o, ffn_w1, ffn_w2):
    N, D = x.shape
    E = edge_index.shape[1]
    H = W_h.shape[0]
    T = 3

    NP = -(-N // 128) * 128
    BE = 8000 if E % 8000 == 0 else E

    src = edge_index[0]
    dst = edge_index[1]
    x_p = jnp.pad(x, ((0, NP - N), (0, 0)))
    batch2 = jnp.pad(batch, (0, NP - N), constant_values=G).reshape(1, NP)
    zeros = jnp.zeros((NP, H), jnp.float32)

    xw = _tc_node_pre(x_p, W_i[:D])
    xs = _sc_gather(xw, src)
    h0, g = _tc_edge_init(xs, edge_attr, W_i[D:], W_h, BE)

    for t in range(T):
        s = _tc_merge(_sc_scatter(g, dst, zeros))
        ss = _sc_gather(s, src)
        g = _tc_edge_step(h0, ss, g, W_h, BE, last=(t == T - 1))

    a2 = _sc_scatter(g, dst, zeros)
    preds = _tc_head(x_p, a2, batch2, W_o[:D], W_o[D:], ffn_w1, ffn_w2)
    return preds.reshape(-1)


# speedup vs baseline: 3.9154x; 1.2631x over previous
import jax
import jax.numpy as jnp
from jax import lax
from jax.experimental import pallas as pl
from jax.experimental.pallas import tpu as pltpu
from jax.experimental.pallas import tpu_sc as plsc

NC = 2
NS = 16
NW = NC * NS
G = 64


def _sc_mesh():
    return plsc.VectorSubcoreMesh(
        core_axis_name="c", subcore_axis_name="s", num_cores=NC, num_subcores=NS
    )


def _chunk(per_w):
    for c in range(min(80, per_w), 7, -1):
        if per_w % c == 0 and c % 8 == 0:
            return c
    raise ValueError(f"no valid chunk for {per_w}")


def _sc_gather(table, idx):
    NP, H = table.shape
    E = idx.shape[0]
    per_w = E // NW
    C = _chunk(per_w)
    stripe = NP // NS

    nch = per_w // C

    def body(table_hbm, idx_hbm, out_hbm, s_sh,
             i0, i1, r0, r1, si0, si1, ss0, ss1, sg):
        cid = lax.axis_index("c")
        sid = lax.axis_index("s")
        wid = sid * NC + cid
        w0 = wid * per_w
        pltpu.async_copy(idx_hbm.at[pl.ds(w0, C)], i0, si0)
        pltpu.async_copy(idx_hbm.at[pl.ds(w0 + C, C)], i1, si1)
        pltpu.sync_copy(table_hbm.at[pl.ds(sid * stripe, stripe)],
                        s_sh.at[pl.ds(sid * stripe, stripe)])
        plsc.subcore_barrier()

        def do_chunk(k, bi, br, semi, sems):
            base = w0 + k * C
            pltpu.make_async_copy(idx_hbm.at[pl.ds(0, C)], bi, semi).wait()

            @pl.when(k >= 2)
            def _():
                pltpu.make_async_copy(br, out_hbm.at[pl.ds(0, C)], sems).wait()

            pltpu.async_copy(s_sh.at[bi], br, sg).wait()
            pltpu.async_copy(br, out_hbm.at[pl.ds(base, C)], sems)

            @pl.when(k + 2 < nch)
            def _():
                pltpu.async_copy(idx_hbm.at[pl.ds(base + 2 * C, C)], bi, semi)

        def pair(j, carry):
            do_chunk(2 * j, i0, r0, si0, ss0)
            do_chunk(2 * j + 1, i1, r1, si1, ss1)
            return carry

        lax.fori_loop(0, nch // 2, pair, 0)
        if nch % 2:
            do_chunk(nch - 1, i0, r0, si0, ss0)
        if nch >= 2:
            pltpu.make_async_copy(r1, out_hbm.at[pl.ds(0, C)], ss1).wait()
        pltpu.make_async_copy(r0, out_hbm.at[pl.ds(0, C)], ss0).wait()

    return pl.kernel(
        body,
        out_type=jax.ShapeDtypeStruct((E, H), jnp.float32),
        mesh=_sc_mesh(),
        scratch_types=[
            pltpu.VMEM_SHARED((NP, H), jnp.float32),
            pltpu.VMEM((C,), jnp.int32),
            pltpu.VMEM((C,), jnp.int32),
            pltpu.VMEM((C, H), jnp.float32),
            pltpu.VMEM((C, H), jnp.float32),
            pltpu.SemaphoreType.DMA,
            pltpu.SemaphoreType.DMA,
            pltpu.SemaphoreType.DMA,
            pltpu.SemaphoreType.DMA,
            pltpu.SemaphoreType.DMA,
        ],
    )(table, idx)


def _sc_scatter(vals, dst, zeros):
    E, H = vals.shape
    NP = zeros.shape[0]
    per_w = E // NW
    C = _chunk(per_w)
    stripe = NP // NS

    nch = per_w // C

    def body(vals_hbm, dst_hbm, zero_hbm, out_hbm, s_sh,
             i0, i1, v0, v1, si0, si1, sv0, sv1):
        cid = lax.axis_index("c")
        sid = lax.axis_index("s")
        wid = sid * NC + cid
        w0 = wid * per_w
        pltpu.async_copy(dst_hbm.at[pl.ds(w0, C)], i0, si0)
        pltpu.async_copy(vals_hbm.at[pl.ds(w0, C)], v0, sv0)
        pltpu.async_copy(dst_hbm.at[pl.ds(w0 + C, C)], i1, si1)
        pltpu.async_copy(vals_hbm.at[pl.ds(w0 + C, C)], v1, sv1)
        pltpu.sync_copy(zero_hbm.at[pl.ds(sid * stripe, stripe)],
                        s_sh.at[pl.ds(sid * stripe, stripe)])
        plsc.subcore_barrier()

        def do_chunk(k, bi, bv, semi, semv):
            base = w0 + k * C
            pltpu.make_async_copy(dst_hbm.at[pl.ds(0, C)], bi, semi).wait()
            pltpu.make_async_copy(vals_hbm.at[pl.ds(0, C)], bv, semv).wait()
            pltpu.sync_copy(bv, s_sh.at[bi], add=True)

            @pl.when(k + 2 < nch)
            def _():
                pltpu.async_copy(dst_hbm.at[pl.ds(base + 2 * C, C)], bi, semi)
                pltpu.async_copy(vals_hbm.at[pl.ds(base + 2 * C, C)], bv, semv)

        def pair(j, carry):
            do_chunk(2 * j, i0, v0, si0, sv0)
            do_chunk(2 * j + 1, i1, v1, si1, sv1)
            return carry

        lax.fori_loop(0, nch // 2, pair, 0)
        if nch % 2:
            do_chunk(nch - 1, i0, v0, si0, sv0)
        plsc.subcore_barrier()
        pltpu.sync_copy(s_sh.at[pl.ds(sid * stripe, stripe)],
                        out_hbm.at[cid, pl.ds(sid * stripe, stripe)])

    return pl.kernel(
        body,
        out_type=jax.ShapeDtypeStruct((NC, NP, H), jnp.float32),
        mesh=_sc_mesh(),
        scratch_types=[
            pltpu.VMEM_SHARED((NP, H), jnp.float32),
            pltpu.VMEM((C,), jnp.int32),
            pltpu.VMEM((C,), jnp.int32),
            pltpu.VMEM((C, H), jnp.float32),
            pltpu.VMEM((C, H), jnp.float32),
            pltpu.SemaphoreType.DMA,
            pltpu.SemaphoreType.DMA,
            pltpu.SemaphoreType.DMA,
            pltpu.SemaphoreType.DMA,
        ],
    )(vals, dst, zeros)


def _tc_node_pre(x_p, W1):
    NP, _ = x_p.shape
    H = W1.shape[1]

    def body(x_ref, w_ref, out_ref):
        out_ref[...] = jnp.dot(x_ref[...], w_ref[...],
                               preferred_element_type=jnp.float32)

    return pl.pallas_call(
        body, out_shape=jax.ShapeDtypeStruct((NP, H), jnp.float32)
    )(x_p, W1)


def _tc_merge(parts):
    _, NP, H = parts.shape

    def body(p_ref, out_ref):
        out_ref[...] = p_ref[0] + p_ref[1]

    return pl.pallas_call(
        body, out_shape=jax.ShapeDtypeStruct((NP, H), jnp.float32)
    )(parts)


def _tc_edge_init(xs, ea, W2, Wh, BE):
    E, H = xs.shape
    DE = ea.shape[1]

    def body(xs_ref, ea_ref, w2_ref, wh_ref, h0_ref, g_ref):
        h0 = jax.nn.relu(
            xs_ref[...]
            + jnp.dot(ea_ref[...], w2_ref[...], preferred_element_type=jnp.float32)
        )
        h0_ref[...] = h0
        g_ref[...] = jnp.dot(h0, wh_ref[...], preferred_element_type=jnp.float32)

    return pl.pallas_call(
        body,
        grid=(E // BE,),
        in_specs=[
            pl.BlockSpec((BE, H), lambda i: (i, 0)),
            pl.BlockSpec((BE, DE), lambda i: (i, 0)),
            pl.BlockSpec((DE, H), lambda i: (0, 0)),
            pl.BlockSpec((H, H), lambda i: (0, 0)),
        ],
        out_specs=[
            pl.BlockSpec((BE, H), lambda i: (i, 0)),
            pl.BlockSpec((BE, H), lambda i: (i, 0)),
        ],
        out_shape=[
            jax.ShapeDtypeStruct((E, H), jnp.float32),
            jax.ShapeDtypeStruct((E, H), jnp.float32),
        ],
    )(xs, ea, W2, Wh)


def _pair_swap(g):
    up = jnp.roll(g, -1, axis=0)
    down = jnp.roll(g, 1, axis=0)
    par = lax.broadcasted_iota(jnp.int32, g.shape, 0) % 2
    return jnp.where(par == 0, up, down)


def _tc_edge_step(h0, ss, g, Wh, BE, last):
    E, H = h0.shape

    def body(h0_ref, ss_ref, g_ref, wh_ref, out_ref):
        h = jax.nn.relu(h0_ref[...] + ss_ref[...] - _pair_swap(g_ref[...]))
        if not last:
            h = jnp.dot(h, wh_ref[...], preferred_element_type=jnp.float32)
        out_ref[...] = h

    return pl.pallas_call(
        body,
        grid=(E // BE,),
        in_specs=[
            pl.BlockSpec((BE, H), lambda i: (i, 0)),
            pl.BlockSpec((BE, H), lambda i: (i, 0)),
            pl.BlockSpec((BE, H), lambda i: (i, 0)),
            pl.BlockSpec((H, H), lambda i: (0, 0)),
        ],
        out_specs=pl.BlockSpec((BE, H), lambda i: (i, 0)),
        out_shape=jax.ShapeDtypeStruct((E, H), jnp.float32),
    )(h0, ss, g, Wh)


def _tc_head(x_p, a2, batch2, Wo1, Wo2, f1, f2):
    NP, _ = x_p.shape

    def body(x_ref, a_ref, b_ref, wo1_ref, wo2_ref, f1_ref, f2_ref, out_ref):
        a = a_ref[0] + a_ref[1]
        nh = jax.nn.relu(
            jnp.dot(x_ref[...], wo1_ref[...], preferred_element_type=jnp.float32)
            + jnp.dot(a, wo2_ref[...], preferred_element_type=jnp.float32)
        )
        gid = lax.broadcasted_iota(jnp.int32, (G, NP), 0)
        oh = (b_ref[...] == gid).astype(jnp.float32)
        cnt = jnp.maximum(jnp.sum(oh, axis=1), 1.0)
        gh = jnp.dot(oh, nh, preferred_element_type=jnp.float32) / cnt[:, None]
        hid = jax.nn.relu(jnp.dot(gh, f1_ref[...], preferred_element_type=jnp.float32))
        out_ref[...] = jnp.dot(hid, f2_ref[...], preferred_element_type=jnp.float32)

    return pl.pallas_call(
        body, out_shape=jax.ShapeDtypeStruct((G, 1), jnp.float32)
    )(x_p, a2, batch2, Wo1, Wo2, f1, f2)


def kernel(x, edge_index, edge_attr, batch, W_i, W_h, W_o, ffn_w1, ffn_w2):
    N, D = x.shape
    E = edge_index.shape[1]
    H = W_h.shape[0]
    T = 3

    NP = -(-N // 128) * 128
    BE = 8000 if E % 8000 == 0 else E

    src = edge_index[0]
    dst = edge_index[1]
    x_p = jnp.pad(x, ((0, NP - N), (0, 0)))
    batch2 = jnp.pad(batch, (0, NP - N), constant_values=G).reshape(1, NP)
    zeros = jnp.zeros((NP, H), jnp.float32)

    xw = _tc_node_pre(x_p, W_i[:D])
    xs = _sc_gather(xw, src)
    h0, g = _tc_edge_init(xs, edge_attr, W_i[D:], W_h, BE)

    for t in range(T):
        s = _tc_merge(_sc_scatter(g, dst, zeros))
        ss = _sc_gather(s, src)
        g = _tc_edge_step(h0, ss, g, W_h, BE, last=(t == T - 1))

    a2 = _sc_scatter(g, dst, zeros)
    preds = _tc_head(x_p, a2, batch2, W_o[:D], W_o[D:], ffn_w1, ffn_w2)
    return preds.reshape(-1)
